# baseline (speedup 1.0000x reference)
import functools

import jax
import jax.numpy as jnp
from jax import lax
from jax.experimental import pallas as pl
from jax.experimental.pallas import tpu as pltpu
from jax.experimental.pallas import tpu_sc as plsc
from jax.sharding import Mesh, PartitionSpec as P

_CHUNK = 16
_NBUF = 4
_AHEAD = 2


def _make_gather(n_rows, d_model, n_workers, num_cores, chunk):
    n_per_w = n_rows // n_workers
    n_chunks = n_per_w // chunk
    assert n_per_w % chunk == 0 and n_chunks % _NBUF == 0
    n_groups = n_chunks // _NBUF
    mesh = plsc.VectorSubcoreMesh(core_axis_name="c", subcore_axis_name="s")

    @functools.partial(
        pl.kernel,
        mesh=mesh,
        out_type=jax.ShapeDtypeStruct((n_rows, d_model), jnp.float32),
        scratch_types=[
            pltpu.VMEM((n_chunks, chunk), jnp.int32),
            pltpu.VMEM((_NBUF, chunk, d_model), jnp.float32),
        ]
        + [pltpu.SemaphoreType.DMA] * (2 * _NBUF),
    )
    def gather_kernel(table_hbm, idx_hbm, out_hbm, idx_v, rows_v, *sems):
        gsem = sems[:_NBUF]
        ssem = sems[_NBUF:]
        wid = lax.axis_index("s") * num_cores + lax.axis_index("c")
        base = wid * n_per_w

        pltpu.sync_copy(idx_hbm.at[wid], idx_v)

        def start_gather(b, g):
            pltpu.async_copy(table_hbm.at[idx_v.at[g]], rows_v.at[b], gsem[b])

        def wait_gather(b, g):
            pltpu.make_async_copy(
                table_hbm.at[idx_v.at[g]], rows_v.at[b], gsem[b]
            ).wait()

        def start_scatter(b, g):
            pltpu.async_copy(
                rows_v.at[b], out_hbm.at[pl.ds(base + g * chunk, chunk)], ssem[b]
            )

        def wait_scatter(b):
            pltpu.make_async_copy(
                rows_v.at[b], out_hbm.at[pl.ds(base, chunk)], ssem[b]
            ).wait()

        for b in range(_AHEAD):
            start_gather(b, b)

        def group(o, carry):
            for j in range(_NBUF):
                g = o * _NBUF + j
                bn = (j + _AHEAD) % _NBUF
                wait_gather(j, g)
                start_scatter(j, g)

                @pl.when(g + _AHEAD < n_chunks)
                def _():
                    @pl.when(g >= _NBUF - _AHEAD)
                    def _():
                        wait_scatter(bn)

                    start_gather(bn, g + _AHEAD)

            return carry

        lax.fori_loop(0, n_groups, group, 0)

        for j in range(_NBUF):
            wait_scatter(j)

    return gather_kernel


def _shard_gather(pe, token_positions):
    batch, seq_len = token_positions.shape
    d_model = pe.shape[1]
    n_rows = batch * seq_len

    info = plsc.get_sparse_core_info()
    n_workers = info.num_cores * info.num_subcores
    n_per_w = n_rows // n_workers
    idx = token_positions.reshape(n_workers, n_per_w // _CHUNK, _CHUNK)

    gather = _make_gather(n_rows, d_model, n_workers, info.num_cores, _CHUNK)
    out = gather(pe, idx)
    return out.reshape(batch, seq_len, d_model)


def kernel(pe, token_positions):
    batch = token_positions.shape[0]
    devices = jax.devices()
    n_dev = 2 if (len(devices) >= 2 and batch % 2 == 0) else 1
    mesh = Mesh(devices[:n_dev], ("d",))
    sharded = jax.shard_map(
        _shard_gather,
        mesh=mesh,
        in_specs=(P(None, None), P("d", None)),
        out_specs=P("d", None, None),
        check_vma=False,
    )
    return sharded(pe, token_positions)


# speedup vs baseline: 4.2521x; 4.2521x over previous
import functools

import jax
import jax.numpy as jnp
from jax import lax
from jax.experimental import pallas as pl
from jax.experimental.pallas import tpu as pltpu
from jax.experimental.pallas import tpu_sc as plsc
from jax.sharding import Mesh, PartitionSpec as P

_CHUNK = 16
_NBUF = 4
_AHEAD = 2


def _make_gather(n_rows, d_model, n_workers, num_cores, chunk):
    n_per_w = n_rows // n_workers
    n_chunks = n_per_w // chunk
    assert n_per_w % chunk == 0 and n_chunks % _NBUF == 0
    n_groups = n_chunks // _NBUF
    mesh = plsc.VectorSubcoreMesh(core_axis_name="c", subcore_axis_name="s")

    @functools.partial(
        pl.kernel,
        mesh=mesh,
        out_type=jax.ShapeDtypeStruct((n_rows, d_model), jnp.float32),
        scratch_types=[
            pltpu.VMEM((n_chunks, chunk), jnp.int32),
            pltpu.VMEM((_NBUF, chunk, d_model), jnp.float32),
        ]
        + [pltpu.SemaphoreType.DMA] * (2 * _NBUF),
    )
    def gather_kernel(table_hbm, idx_hbm, out_hbm, idx_v, rows_v, *sems):
        gsem = sems[:_NBUF]
        ssem = sems[_NBUF:]
        wid = lax.axis_index("s") * num_cores + lax.axis_index("c")
        base = wid * n_per_w

        pltpu.sync_copy(idx_hbm.at[wid], idx_v)

        def start_gather(b, g):
            pltpu.async_copy(table_hbm.at[idx_v.at[g]], rows_v.at[b], gsem[b])

        def wait_gather(b, g):
            pltpu.make_async_copy(
                table_hbm.at[idx_v.at[g]], rows_v.at[b], gsem[b]
            ).wait()

        def start_scatter(b, g):
            pltpu.async_copy(
                rows_v.at[b], out_hbm.at[pl.ds(base + g * chunk, chunk)], ssem[b]
            )

        def wait_scatter(b):
            pltpu.make_async_copy(
                rows_v.at[b], out_hbm.at[pl.ds(base, chunk)], ssem[b]
            ).wait()

        for b in range(_AHEAD):
            start_gather(b, b)

        def group(o, carry):
            for j in range(_NBUF):
                g = o * _NBUF + j
                bn = (j + _AHEAD) % _NBUF

                @pl.when(g + _AHEAD < n_chunks)
                def _():
                    @pl.when(g >= _NBUF - _AHEAD)
                    def _():
                        wait_scatter(bn)

                    start_gather(bn, g + _AHEAD)

                wait_gather(j, g)
                start_scatter(j, g)

            return carry

        lax.fori_loop(0, n_groups, group, 0)

        for j in range(_NBUF):
            wait_scatter(j)

    return gather_kernel


def _shard_gather(pe, token_positions):
    batch, seq_len = token_positions.shape
    d_model = pe.shape[1]
    n_rows = batch * seq_len

    info = plsc.get_sparse_core_info()
    n_workers = info.num_cores * info.num_subcores
    n_per_w = n_rows // n_workers
    idx = token_positions.reshape(n_workers, n_per_w // _CHUNK, _CHUNK)

    gather = _make_gather(n_rows, d_model, n_workers, info.num_cores, _CHUNK)
    out = gather(pe, idx)
    return out.reshape(batch, seq_len, d_model)


def kernel(pe, token_positions):
    return _shard_gather(pe, token_positions)


# speedup vs baseline: 4.2857x; 1.0079x over previous
import functools

import jax
import jax.numpy as jnp
from jax import lax
from jax.experimental import pallas as pl
from jax.experimental.pallas import tpu as pltpu
from jax.experimental.pallas import tpu_sc as plsc
from jax.sharding import Mesh, PartitionSpec as P

_CHUNK = 16
_NBUF = 4
_AHEAD = 2


def _make_gather(n_rows, d_model, n_workers, num_cores, chunk):
    n_per_w = n_rows // n_workers
    n_chunks = n_per_w // chunk
    assert n_per_w % chunk == 0 and n_chunks % _NBUF == 0
    n_groups = n_chunks // _NBUF
    mesh = plsc.VectorSubcoreMesh(core_axis_name="c", subcore_axis_name="s")

    @functools.partial(
        pl.kernel,
        mesh=mesh,
        out_type=jax.ShapeDtypeStruct((n_rows, d_model), jnp.float32),
        scratch_types=[
            pltpu.VMEM((n_chunks, chunk), jnp.int32),
            pltpu.VMEM((_NBUF, chunk, d_model), jnp.float32),
        ]
        + [pltpu.SemaphoreType.DMA] * (2 * _NBUF),
    )
    def gather_kernel(table_hbm, idx_hbm, out_hbm, idx_v, rows_v, *sems):
        gsem = sems[:_NBUF]
        ssem = sems[_NBUF:]
        wid = lax.axis_index("s") * num_cores + lax.axis_index("c")

        pltpu.sync_copy(idx_hbm.at[:, wid], idx_v)

        def start_gather(b, g):
            pltpu.async_copy(table_hbm.at[idx_v.at[g]], rows_v.at[b], gsem[b])

        def wait_gather(b, g):
            pltpu.make_async_copy(
                table_hbm.at[idx_v.at[g]], rows_v.at[b], gsem[b]
            ).wait()

        def start_scatter(b, g):
            off = (g * n_workers + wid) * chunk
            pltpu.async_copy(
                rows_v.at[b], out_hbm.at[pl.ds(off, chunk)], ssem[b]
            )

        def wait_scatter(b):
            pltpu.make_async_copy(
                rows_v.at[b], out_hbm.at[pl.ds(wid * chunk, chunk)], ssem[b]
            ).wait()

        for b in range(_AHEAD):
            start_gather(b, b)

        def group(o, carry):
            for j in range(_NBUF):
                g = o * _NBUF + j
                bn = (j + _AHEAD) % _NBUF

                @pl.when(g + _AHEAD < n_chunks)
                def _():
                    @pl.when(g >= _NBUF - _AHEAD)
                    def _():
                        wait_scatter(bn)

                    start_gather(bn, g + _AHEAD)

                wait_gather(j, g)
                start_scatter(j, g)

            return carry

        lax.fori_loop(0, n_groups, group, 0)

        for j in range(_NBUF):
            wait_scatter(j)

    return gather_kernel


def _shard_gather(pe, token_positions):
    batch, seq_len = token_positions.shape
    d_model = pe.shape[1]
    n_rows = batch * seq_len

    info = plsc.get_sparse_core_info()
    n_workers = info.num_cores * info.num_subcores
    n_per_w = n_rows // n_workers
    idx = token_positions.reshape(n_per_w // _CHUNK, n_workers, _CHUNK)

    gather = _make_gather(n_rows, d_model, n_workers, info.num_cores, _CHUNK)
    out = gather(pe, idx)
    return out.reshape(batch, seq_len, d_model)


def kernel(pe, token_positions):
    return _shard_gather(pe, token_positions)
